# baseline (speedup 1.0000x reference)
import functools

import jax
import jax.numpy as jnp
from jax import lax
from jax.experimental import pallas as pl
from jax.experimental.pallas import tpu as pltpu
from jax.experimental.pallas import tpu_sc as plsc

_MAXLEN = 200
_D = 64
_B = 4096
_NC, _NS = 2, 16
_NW = _NC * _NS
_G = _B // _NW
_NG = _MAXLEN
_NBUF = 5
_XCHUNK = 32


def _body(tok_hbm, x_hbm, pos_hbm, out_hbm, xrow_v, idx_v, pos_v,
          b0, b1, b2, b3, b4, g0, g1, g2, g3, g4, o0, o1, o2, o3, o4):
    bufs = [b0, b1, b2, b3, b4]
    gsems = [g0, g1, g2, g3, g4]
    osems = [o0, o1, o2, o3, o4]

    wid = lax.axis_index("s") * _NC + lax.axis_index("c")
    base = wid * _G
    pltpu.sync_copy(pos_hbm, pos_v)

    rows16 = [lax.iota(jnp.int32, 16) + 16 * k for k in range(_XCHUNK // 16)]
    for c4 in range(_G // _XCHUNK):
        pltpu.sync_copy(x_hbm.at[pl.ds(base + _XCHUNK * c4, _XCHUNK)], xrow_v)

        @plsc.parallel_loop(0, _NG, step=1, unroll=4)
        def _transpose(j, c4=c4):
            col = jnp.full((16,), j, jnp.int32)
            for t in range(_XCHUNK // 16):
                idx_v[j, pl.ds(_XCHUNK * c4 + 16 * t, 16)] = plsc.load_gather(
                    xrow_v, [rows16[t], col])

    def fire_gather(j, b):
        pltpu.async_copy(tok_hbm.at[idx_v.at[j]], bufs[b], gsems[b])

    def wait_gather(j, b):
        pltpu.make_async_copy(tok_hbm.at[idx_v.at[j]], bufs[b], gsems[b]).wait()

    def fire_write(j, b):
        pltpu.async_copy(
            bufs[b].at[:, pl.ds(0, _D)],
            out_hbm.at[pl.ds(base, _G), j, pl.ds(0, _D)], osems[b])

    def wait_write(j, b):
        pltpu.make_async_copy(
            bufs[b].at[:, pl.ds(0, _D)],
            out_hbm.at[pl.ds(base, _G), j, pl.ds(0, _D)], osems[b]).wait()

    def compute(j, b):
        pvs = [pos_v[j, pl.ds(c * 16, 16)] for c in range(_D // 16)]

        @plsc.parallel_loop(0, _G, step=1, unroll=8)
        def add_row(i):
            for c in range(_D // 16):
                sl = pl.ds(c * 16, 16)
                bufs[b][i, sl] = bufs[b][i, sl] + pvs[c]

    fire_gather(0, 0)
    fire_gather(1, 1)

    for j in range(3):
        wait_gather(j, j)
        compute(j, j)
        fire_write(j, j)
        fire_gather(j + 2, j + 2)

    def round_body(r, carry):
        j0 = 3 + r * _NBUF
        for p in range(_NBUF):
            j = j0 + p
            b = (3 + p) % _NBUF
            wait_gather(j, b)
            compute(j, b)
            fire_write(j, b)
            wait_write(j - 3, (b + 2) % _NBUF)
            fire_gather(j + 2, (b + 2) % _NBUF)
        return carry

    lax.fori_loop(0, (_NG - _NBUF) // _NBUF, round_body, 0)

    for j in (198, 199):
        b = j % _NBUF
        wait_gather(j, b)
        compute(j, b)
        fire_write(j, b)

    for k in range(_NBUF):
        wait_write(195 + k, k)


_emb = functools.partial(
    pl.kernel,
    out_type=jax.ShapeDtypeStruct((_B, _MAXLEN, 128), jnp.float32),
    mesh=plsc.VectorSubcoreMesh(
        core_axis_name="c", subcore_axis_name="s",
        num_cores=_NC, num_subcores=_NS),
    scratch_types=(
        [pltpu.VMEM((_XCHUNK, _NG), jnp.int32),
         pltpu.VMEM((_NG, _G), jnp.int32),
         pltpu.VMEM((_MAXLEN, _D), jnp.float32)]
        + [pltpu.VMEM((_G, 128), jnp.float32) for _ in range(_NBUF)]
        + [pltpu.SemaphoreType.DMA for _ in range(2 * _NBUF)]
    ),
    compiler_params=pltpu.CompilerParams(
        use_tc_tiling_on_sc=False, needs_layout_passes=False),
)(_body)


def kernel(x, token_table, pos_table):
    tok_pad = jnp.pad(token_table, ((0, 0), (0, 128 - _D)))
    return _emb(tok_pad, x, pos_table)[..., :_D]


# speedup vs baseline: 1.2426x; 1.2426x over previous
import functools

import jax
import jax.numpy as jnp
from jax import lax
from jax.experimental import pallas as pl
from jax.experimental.pallas import tpu as pltpu
from jax.experimental.pallas import tpu_sc as plsc

_MAXLEN = 200
_D = 64
_B = 4096
_NC, _NS = 2, 16
_NW = _NC * _NS
_G = _B // _NW
_NG = _MAXLEN
_NBUF = 4
_NROUND = _NG // _NBUF
_NCHUNK = 4
_XROWS = _G * _NG // 128 // _NCHUNK


def _body(tok_hbm, x_hbm, pos_hbm, out_hbm, xstage_v, idx_v, pos_v,
          bi0, bi1, bi2, bi3, bo0, bo1, bo2, bo3,
          g0, g1, g2, g3, o0, o1, o2, o3):
    bins = [bi0, bi1, bi2, bi3]
    bouts = [bo0, bo1, bo2, bo3]
    gsems = [g0, g1, g2, g3]
    osems = [o0, o1, o2, o3]

    wid = lax.axis_index("s") * _NC + lax.axis_index("c")
    base = wid * _G
    pltpu.sync_copy(pos_hbm, pos_v)

    rows16 = [lax.iota(jnp.int32, 16) + 16 * t for t in range(2)]
    bpc = _G // _NCHUNK
    for c4 in range(_NCHUNK):
        pltpu.sync_copy(
            x_hbm.at[pl.ds(wid * _NG + _XROWS * c4, _XROWS)], xstage_v)

        @plsc.parallel_loop(0, _NG, step=1, unroll=4)
        def _transpose(j, c4=c4):
            col = jnp.full((16,), j, jnp.int32)
            for t in range(2):
                off = rows16[t] * _NG + col
                idx_v[j, pl.ds(bpc * c4 + 16 * t, 16)] = plsc.load_gather(
                    xstage_v, [off >> 7, off & 127])

    def fire_gather(j, b):
        pltpu.async_copy(tok_hbm.at[idx_v.at[j]], bins[b], gsems[b])

    def wait_gather(j, b):
        pltpu.make_async_copy(tok_hbm.at[idx_v.at[j]], bins[b], gsems[b]).wait()

    def fire_write(j, b):
        pltpu.async_copy(
            bouts[b], out_hbm.at[pl.ds(base, _G), j, pl.ds(0, _D)], osems[b])

    def wait_write(j, b):
        pltpu.make_async_copy(
            bouts[b], out_hbm.at[pl.ds(base, _G), j, pl.ds(0, _D)],
            osems[b]).wait()

    def compute(j, b):
        lane0 = (j & 1) * _D
        pvs = [pos_v[j >> 1, pl.ds(lane0 + c * 16, 16)]
               for c in range(_D // 16)]

        @plsc.parallel_loop(0, _G, step=1, unroll=8)
        def add_row(i):
            for c in range(_D // 16):
                sl = pl.ds(c * 16, 16)
                bouts[b][i, sl] = bins[b][i, sl] + pvs[c]

    for b in range(_NBUF):
        fire_gather(b, b)

    for b in range(_NBUF):
        wait_gather(b, b)
        compute(b, b)
        fire_write(b, b)
        fire_gather(_NBUF + b, b)

    def round_body(r, carry):
        j0 = r * _NBUF
        for b in range(_NBUF):
            j = j0 + b
            wait_gather(j, b)
            wait_write(j - _NBUF, b)
            compute(j, b)
            fire_write(j, b)
            fire_gather(j + _NBUF, b)
        return carry

    lax.fori_loop(1, _NROUND - 1, round_body, 0)

    j0 = (_NROUND - 1) * _NBUF
    for b in range(_NBUF):
        j = j0 + b
        wait_gather(j, b)
        wait_write(j - _NBUF, b)
        compute(j, b)
        fire_write(j, b)

    for b in range(_NBUF):
        wait_write(j0 + b, b)


_emb = functools.partial(
    pl.kernel,
    out_type=jax.ShapeDtypeStruct((_B, _MAXLEN, 128), jnp.float32),
    mesh=plsc.VectorSubcoreMesh(
        core_axis_name="c", subcore_axis_name="s",
        num_cores=_NC, num_subcores=_NS),
    scratch_types=(
        [pltpu.VMEM((_XROWS, 128), jnp.int32),
         pltpu.VMEM((_NG, _G), jnp.int32),
         pltpu.VMEM((_MAXLEN // 2, 128), jnp.float32)]
        + [pltpu.VMEM((_G, _D), jnp.float32) for _ in range(2 * _NBUF)]
        + [pltpu.SemaphoreType.DMA for _ in range(2 * _NBUF)]
    ),
    compiler_params=pltpu.CompilerParams(
        use_tc_tiling_on_sc=False, needs_layout_passes=False),
)(_body)


def kernel(x, token_table, pos_table):
    x2 = jnp.reshape(x, (_B * _MAXLEN // 128, 128))
    pos2 = jnp.reshape(pos_table, (_MAXLEN // 2, 128))
    return _emb(token_table, x2, pos2)[..., :_D]
